# speedup vs baseline: 5.2031x; 5.2031x over previous
import jax
import jax.numpy as jnp
from jax.experimental import pallas as pl
from jax.experimental.pallas import tpu as pltpu

_B = 16
_BLK = 2048


def _seg_mean_body(ids_ref, x_ref, out_ref, cnt_ref):
    i = pl.program_id(0)

    @pl.when(i == 0)
    def _init():
        out_ref[...] = jnp.zeros_like(out_ref)
        cnt_ref[...] = jnp.zeros_like(cnt_ref)

    ids = ids_ref[0, 0, :]
    onehot = (
        ids[:, None] == jax.lax.broadcasted_iota(jnp.int32, (_BLK, _B), 1)
    ).astype(jnp.float32)
    out_ref[...] += jax.lax.dot_general(
        onehot, x_ref[...], (((0,), (0,)), ((), ())),
        preferred_element_type=jnp.float32,
    )
    cnt_ref[...] += jnp.sum(onehot, axis=0, keepdims=True)

    @pl.when(i == pl.num_programs(0) - 1)
    def _finish():
        out_ref[...] = out_ref[...] / jnp.maximum(cnt_ref[0, :], 1.0)[:, None]


def kernel(flat, segment_ids):
    total, d = flat.shape
    n_blocks = total // _BLK
    ids3 = segment_ids.astype(jnp.int32).reshape(n_blocks, 1, _BLK)
    grid = (n_blocks,)
    return pl.pallas_call(
        _seg_mean_body,
        grid=grid,
        in_specs=[
            pl.BlockSpec((1, 1, _BLK), lambda i: (i, 0, 0)),
            pl.BlockSpec((_BLK, d), lambda i: (i, 0)),
        ],
        out_specs=pl.BlockSpec((_B, d), lambda i: (0, 0)),
        out_shape=jax.ShapeDtypeStruct((_B, d), jnp.float32),
        scratch_shapes=[pltpu.VMEM((1, _B), jnp.float32)],
        compiler_params=pltpu.CompilerParams(
            dimension_semantics=("arbitrary",),
        ),
    )(ids3, flat)
